# baseline (speedup 1.0000x reference)
import functools

import jax
import jax.numpy as jnp
from jax import lax
from jax.experimental import pallas as pl
from jax.experimental.pallas import tpu as pltpu
from jax.experimental.pallas import tpu_sc as plsc

NW = 32
CHUNK = 128
RING = 4
ROWS_PER_TILE = 632
N_PAD = 16 * ROWS_PER_TILE


def _tc_transform(x, W, b):
    N, Din = x.shape
    R, _, Dh = W.shape

    def body(x_ref, w_ref, b_ref, out_ref):
        h = jnp.dot(x_ref[...], w_ref[0], preferred_element_type=jnp.float32)
        out_ref[0] = jnp.maximum(h + b_ref[...], 0.0)

    return pl.pallas_call(
        body,
        grid=(R,),
        in_specs=[
            pl.BlockSpec((N, Din), lambda r: (0, 0)),
            pl.BlockSpec((1, Din, Dh), lambda r: (r, 0, 0)),
            pl.BlockSpec((1, Dh), lambda r: (0, 0)),
        ],
        out_specs=pl.BlockSpec((1, N, Dh), lambda r: (r, 0, 0)),
        out_shape=jax.ShapeDtypeStruct((R, N, Dh), jnp.float32),
    )(x, W, b.reshape(1, Dh))


def _tc_transform_sum(parts, W, b, N):
    R, _, Dh = W.shape
    Din = parts.shape[2]

    def body(p_ref, w_ref, b_ref, out_ref):
        h = p_ref[0] + p_ref[1]
        hh = jnp.dot(h, w_ref[0], preferred_element_type=jnp.float32)
        out_ref[0] = jnp.maximum(hh + b_ref[...], 0.0)

    return pl.pallas_call(
        body,
        grid=(R,),
        in_specs=[
            pl.BlockSpec((2, N, Din), lambda r: (0, 0, 0)),
            pl.BlockSpec((1, Din, Dh), lambda r: (r, 0, 0)),
            pl.BlockSpec((1, Dh), lambda r: (0, 0)),
        ],
        out_specs=pl.BlockSpec((1, N, Dh), lambda r: (r, 0, 0)),
        out_shape=jax.ShapeDtypeStruct((R, N, Dh), jnp.float32),
    )(parts, W, b.reshape(1, Dh))


def _tc_sum(parts, N):
    D = parts.shape[2]

    def body(p_ref, out_ref):
        out_ref[...] = p_ref[0] + p_ref[1]

    return pl.pallas_call(
        body,
        grid=(1,),
        in_specs=[pl.BlockSpec((2, N, D), lambda i: (0, 0, 0))],
        out_specs=pl.BlockSpec((N, D), lambda i: (0, 0)),
        out_shape=jax.ShapeDtypeStruct((N, D), jnp.float32),
    )(parts)


def _sc_edge_agg(h_table, pidx, zblock, ct, D):
    mesh = plsc.VectorSubcoreMesh(core_axis_name="c", subcore_axis_name="s")

    @functools.partial(
        pl.kernel,
        mesh=mesh,
        out_type=jax.ShapeDtypeStruct((2, N_PAD, D), jnp.float32),
        scratch_types=[
            pltpu.VMEM((ct + 8, CHUNK), jnp.int32),
            pltpu.VMEM((8, CHUNK), jnp.int32),
            pltpu.VMEM((8, CHUNK), jnp.int32),
            pltpu.VMEM((8, CHUNK), jnp.int32),
            pltpu.VMEM((8, CHUNK), jnp.int32),
            pltpu.VMEM((CHUNK, D), jnp.float32),
            pltpu.VMEM((CHUNK, D), jnp.float32),
            pltpu.VMEM_SHARED((N_PAD, D), jnp.float32),
            pltpu.SemaphoreType.DMA,
            pltpu.SemaphoreType.DMA,
        ],
    )
    def run(h_hbm, pidx_hbm, z_hbm, out_hbm,
            pidx_v, gb0, gb1, db0, db1, buf0, buf1, acc, sem0, sem1):
        cid = lax.axis_index("c")
        sid = lax.axis_index("s")
        base = (sid * 2 + cid) * ct

        pltpu.sync_copy(
            z_hbm, acc.at[pl.ds(sid * ROWS_PER_TILE, ROWS_PER_TILE)])
        pltpu.sync_copy(pidx_hbm.at[pl.ds(base, ct + 8)], pidx_v)
        plsc.subcore_barrier()

        def unpack(j, gb, db):
            for k in range(CHUNK // 16):
                v = pidx_v[j, pl.ds(k * 16, 16)]
                gb[0, pl.ds(k * 16, 16)] = lax.shift_right_logical(v, 14)
                db[0, pl.ds(k * 16, 16)] = lax.bitwise_and(v, 16383)

        def fire_g(gb, buf, sem):
            pltpu.async_copy(h_hbm.at[gb.at[0]], buf, sem)

        def drain_g(gb, buf, sem):
            pltpu.make_async_copy(h_hbm.at[gb.at[0]], buf, sem).wait()

        def scatter(db, buf):
            pltpu.sync_copy(buf, acc.at[db.at[0]], add=True)

        unpack(0, gb0, db0)
        fire_g(gb0, buf0, sem0)

        def body(p, carry):
            j = 2 * p
            unpack(j + 1, gb1, db1)
            drain_g(gb0, buf0, sem0)
            fire_g(gb1, buf1, sem1)
            scatter(db0, buf0)
            unpack(j + 2, gb0, db0)
            drain_g(gb1, buf1, sem1)
            fire_g(gb0, buf0, sem0)
            scatter(db1, buf1)
            return carry

        lax.fori_loop(0, ct // 2, body, 0)
        drain_g(gb0, buf0, sem0)
        plsc.subcore_barrier()

        pltpu.sync_copy(
            acc.at[pl.ds(sid * ROWS_PER_TILE, ROWS_PER_TILE)],
            out_hbm.at[cid, pl.ds(sid * ROWS_PER_TILE, ROWS_PER_TILE)])

    return run(h_table, pidx, zblock)


def kernel(x, edge_index, edge_type, W1, b1, W2, b2):
    N, D = x.shape
    E = edge_index.shape[1]

    src = edge_index[0].astype(jnp.int32)
    dst = edge_index[1].astype(jnp.int32)
    et = edge_type.astype(jnp.int32)

    gidx = et * N + src
    ct = -(-E // (NW * CHUNK))
    ct += ct % 2
    pad = ct * NW * CHUNK - E
    extra = 8 * CHUNK
    gidx = jnp.concatenate([gidx, jnp.zeros((pad + extra,), jnp.int32)])
    didx = jnp.concatenate([dst, jnp.full((pad + extra,), N, jnp.int32)])
    pidx = (gidx * 16384 + didx).reshape(-1, CHUNK)
    zblock = jnp.zeros((ROWS_PER_TILE, D), jnp.float32)

    H1 = _tc_transform(x, W1, b1).reshape(-1, D)
    parts1 = _sc_edge_agg(H1, pidx, zblock, ct, D)
    H2 = _tc_transform_sum(parts1, W2, b2, N).reshape(-1, D)
    parts2 = _sc_edge_agg(H2, pidx, zblock, ct, D)
    return _tc_sum(parts2, N)


# speedup vs baseline: 1.3952x; 1.3952x over previous
import functools

import jax
import jax.numpy as jnp
from jax import lax
from jax.experimental import pallas as pl
from jax.experimental.pallas import tpu as pltpu
from jax.experimental.pallas import tpu_sc as plsc

NW = 32
CHUNK = 128
ZROWS = 64
ROWS_PER_TILE = 640
N_PAD = 16 * ROWS_PER_TILE


def _tc_transform(x, W, b):
    N, Din = x.shape
    R, _, Dh = W.shape

    def body(x_ref, w_ref, b_ref, out_ref):
        h = jnp.dot(x_ref[...], w_ref[0], preferred_element_type=jnp.float32)
        out_ref[0] = jnp.maximum(h + b_ref[...], 0.0)

    return pl.pallas_call(
        body,
        grid=(R,),
        in_specs=[
            pl.BlockSpec((N, Din), lambda r: (0, 0)),
            pl.BlockSpec((1, Din, Dh), lambda r: (r, 0, 0)),
            pl.BlockSpec((1, Dh), lambda r: (0, 0)),
        ],
        out_specs=pl.BlockSpec((1, N, Dh), lambda r: (r, 0, 0)),
        out_shape=jax.ShapeDtypeStruct((R, N, Dh), jnp.float32),
    )(x, W, b.reshape(1, Dh))


def _tc_transform_sum(parts, W, b, N):
    R, _, Dh = W.shape
    Din = parts.shape[2]

    def body(p_ref, w_ref, b_ref, out_ref):
        h = p_ref[0] + p_ref[1]
        hh = jnp.dot(h, w_ref[0], preferred_element_type=jnp.float32)
        out_ref[0] = jnp.maximum(hh + b_ref[...], 0.0)

    return pl.pallas_call(
        body,
        grid=(R,),
        in_specs=[
            pl.BlockSpec((2, N, Din), lambda r: (0, 0, 0)),
            pl.BlockSpec((1, Din, Dh), lambda r: (r, 0, 0)),
            pl.BlockSpec((1, Dh), lambda r: (0, 0)),
        ],
        out_specs=pl.BlockSpec((1, N, Dh), lambda r: (r, 0, 0)),
        out_shape=jax.ShapeDtypeStruct((R, N, Dh), jnp.float32),
    )(parts, W, b.reshape(1, Dh))


def _tc_sum(parts, N):
    D = parts.shape[2]

    def body(p_ref, out_ref):
        out_ref[...] = p_ref[0] + p_ref[1]

    return pl.pallas_call(
        body,
        grid=(1,),
        in_specs=[pl.BlockSpec((2, N, D), lambda i: (0, 0, 0))],
        out_specs=pl.BlockSpec((N, D), lambda i: (0, 0)),
        out_shape=jax.ShapeDtypeStruct((N, D), jnp.float32),
    )(parts)


def _sc_edge_agg(h_table, gidx, didx, zblock, nchunk, D):
    mesh = plsc.VectorSubcoreMesh(core_axis_name="c", subcore_axis_name="s")

    @functools.partial(
        pl.kernel,
        mesh=mesh,
        out_type=jax.ShapeDtypeStruct((2, N_PAD, D), jnp.float32),
        scratch_types=[
            pltpu.VMEM((nchunk, CHUNK), jnp.int32),
            pltpu.VMEM((nchunk, CHUNK), jnp.int32),
            pltpu.VMEM((CHUNK, D), jnp.float32),
            pltpu.VMEM((ZROWS, D), jnp.float32),
            pltpu.VMEM_SHARED((N_PAD, D), jnp.float32),
            pltpu.SemaphoreType.DMA,
        ],
    )
    def run(h_hbm, gidx_hbm, didx_hbm, z_hbm, out_hbm,
            gidx_v, didx_v, rows_v, zbuf, acc, sem):
        cid = lax.axis_index("c")
        sid = lax.axis_index("s")
        wid = sid * 2 + cid

        pltpu.sync_copy(z_hbm, zbuf)
        for k in range(ROWS_PER_TILE // ZROWS):
            pltpu.sync_copy(
                zbuf, acc.at[pl.ds(sid * ROWS_PER_TILE + k * ZROWS, ZROWS)])

        pltpu.sync_copy(gidx_hbm.at[wid], gidx_v)
        pltpu.sync_copy(didx_hbm.at[wid], didx_v)
        plsc.subcore_barrier()

        def body(j, carry):
            pltpu.async_copy(h_hbm.at[gidx_v.at[j]], rows_v, sem).wait()
            pltpu.sync_copy(rows_v, acc.at[didx_v.at[j]], add=True)
            return carry

        lax.fori_loop(0, nchunk, body, 0)
        plsc.subcore_barrier()

        pltpu.sync_copy(
            acc.at[pl.ds(sid * ROWS_PER_TILE, ROWS_PER_TILE)],
            out_hbm.at[cid, pl.ds(sid * ROWS_PER_TILE, ROWS_PER_TILE)])

    return run(h_table, gidx, didx, zblock)


def kernel(x, edge_index, edge_type, W1, b1, W2, b2):
    N, D = x.shape
    E = edge_index.shape[1]

    src = edge_index[0].astype(jnp.int32)
    dst = edge_index[1].astype(jnp.int32)
    et = edge_type.astype(jnp.int32)

    gidx = et * N + src
    ep_total = ((E + NW * CHUNK - 1) // (NW * CHUNK)) * (NW * CHUNK)
    pad = ep_total - E
    nchunk = ep_total // (NW * CHUNK)
    gidx = jnp.concatenate([gidx, jnp.zeros((pad,), jnp.int32)])
    didx = jnp.concatenate([dst, jnp.full((pad,), N, jnp.int32)])
    gidx = gidx.reshape(NW, nchunk, CHUNK)
    didx = didx.reshape(NW, nchunk, CHUNK)
    zblock = jnp.zeros((ZROWS, D), jnp.float32)

    H1 = _tc_transform(x, W1, b1).reshape(-1, D)
    parts1 = _sc_edge_agg(H1, gidx, didx, zblock, nchunk, D)
    H2 = _tc_transform_sum(parts1, W2, b2, N).reshape(-1, D)
    parts2 = _sc_edge_agg(H2, gidx, didx, zblock, nchunk, D)
    return _tc_sum(parts2, N)
